# baseline (speedup 1.0000x reference)
import jax
import jax.numpy as jnp
from jax.experimental import pallas as pl

_B, _S, _H = 2, 2048, 1024
_MS, _CR = 512, 4
_BLK = 256
_CBLK = _BLK // _CR


def _silu(x):
    return x * jax.nn.sigmoid(x)


def _fused_body(hs_ref, hsc_ref,
                rg_w1a_ref, rg_c1_ref, rg_w2t_ref, rg_b2_ref,
                wg_w1_ref, wg_b1_ref, wg_w2t_ref, wg_b2_ref,
                comp_w1_ref, comp_b1_ref, comp_w2_ref, comp_b2_ref,
                o_b_ref, ln_g_ref, ln_b_ref,
                out_ref, bank_ref):
    x = hs_ref[...]
    xb = x.astype(jnp.bfloat16)

    a1 = _silu(jnp.dot(xb, rg_w1a_ref[...], preferred_element_type=jnp.float32)
               + rg_c1_ref[...])
    rlogit = jnp.sum(a1 * rg_w2t_ref[...], axis=-1, keepdims=True) + rg_b2_ref[...]
    rgate = jax.nn.sigmoid(rlogit)
    out = x + rgate * o_b_ref[...]
    mu = jnp.mean(out, axis=-1, keepdims=True)
    d = out - mu
    var = jnp.mean(d * d, axis=-1, keepdims=True)
    out_ref[...] = d * jax.lax.rsqrt(var + 1e-5) * ln_g_ref[...] + ln_b_ref[...]

    a2 = _silu(jnp.dot(xb, wg_w1_ref[...], preferred_element_type=jnp.float32)
               + wg_b1_ref[...])
    wlogit = jnp.sum(a2 * wg_w2t_ref[...], axis=-1, keepdims=True) + wg_b2_ref[...]
    wgate = jax.nn.sigmoid(wlogit)
    row = jax.lax.broadcasted_iota(jnp.int32, (_CBLK, _BLK), 0)
    col = jax.lax.broadcasted_iota(jnp.int32, (_CBLK, _BLK), 1)
    pool = jnp.where(col // _CR == row, 1.0 / _CR, 0.0)
    gmean = jnp.dot(pool, wgate, preferred_element_type=jnp.float32)

    xc = hsc_ref[...].astype(jnp.bfloat16)
    h1 = _silu(jnp.dot(xc, comp_w1_ref[...], preferred_element_type=jnp.float32)
               + comp_b1_ref[...])
    comp = (jnp.dot(h1.astype(jnp.bfloat16), comp_w2_ref[...],
                    preferred_element_type=jnp.float32)
            + comp_b2_ref[...])
    bank_ref[...] = comp * gmean


def kernel(hidden_states, comp_w1, comp_b1, comp_w2, comp_b2,
           q_w, q_b, k_w, k_b, v_w, v_b, o_w, o_b,
           wg_w1, wg_b1, wg_w2, wg_b2, rg_w1, rg_b1, rg_w2, rg_b2,
           ln_g, ln_b):
    b, s, h = hidden_states.shape
    n_tok = b * s
    hs2 = hidden_states.reshape(n_tok, h)
    hsc = hidden_states.reshape(n_tok // _CR, _CR * h)

    rg_w1a = rg_w1[:h].astype(jnp.bfloat16)
    rg_c1 = (o_b @ rg_w1[h:] + rg_b1).reshape(1, -1)

    grid = (n_tok // _BLK,)
    full = lambda arr: pl.BlockSpec(arr.shape, lambda i: (0,) * arr.ndim)

    w_args = (rg_w1a, rg_c1, rg_w2.reshape(1, -1), rg_b2.reshape(1, 1),
              wg_w1.astype(jnp.bfloat16), wg_b1.reshape(1, -1),
              wg_w2.reshape(1, -1), wg_b2.reshape(1, 1),
              comp_w1.astype(jnp.bfloat16), comp_b1.reshape(1, -1),
              comp_w2.astype(jnp.bfloat16), comp_b2.reshape(1, -1),
              o_b.reshape(1, -1), ln_g.reshape(1, -1), ln_b.reshape(1, -1))

    out2, bank2 = pl.pallas_call(
        _fused_body,
        grid=grid,
        in_specs=[pl.BlockSpec((_BLK, h), lambda i: (i, 0)),
                  pl.BlockSpec((_CBLK, _CR * h), lambda i: (i, 0)),
                  *(full(a) for a in w_args)],
        out_specs=[pl.BlockSpec((_BLK, h), lambda i: (i, 0)),
                   pl.BlockSpec((_CBLK, h), lambda i: (i, 0))],
        out_shape=[jax.ShapeDtypeStruct((n_tok, h), jnp.float32),
                   jax.ShapeDtypeStruct((n_tok // _CR, h), jnp.float32)],
    )(hs2, hsc, *w_args)

    output = out2.reshape(b, s, h)
    memory_bank = bank2.reshape(b, s // _CR, h)
    memory_mask = jnp.ones((b, _MS), dtype=bool)
    return (output, memory_bank, memory_mask)


# speedup vs baseline: 1.3541x; 1.3541x over previous
import jax
import jax.numpy as jnp
from jax.experimental import pallas as pl

_B, _S, _H = 2, 2048, 1024
_MS, _CR = 512, 4
_BLK = 512
_CBLK = _BLK // _CR


def _silu(x):
    return x * jax.nn.sigmoid(x)


def _fused_body(hs_ref, hsc_ref,
                rg_w1a_ref, rg_c1_ref, rg_w2t_ref, rg_b2_ref,
                wg_w1_ref, wg_b1_ref, wg_w2t_ref, wg_b2_ref,
                comp_w1_ref, comp_b1_ref, comp_w2_ref, comp_b2_ref,
                o_b_ref, ln_g_ref, ln_b_ref,
                out_ref, bank_ref):
    x = hs_ref[...]

    a1 = _silu(jnp.dot(x, rg_w1a_ref[...], preferred_element_type=jnp.float32)
               + rg_c1_ref[...])
    rlogit = jnp.sum(a1 * rg_w2t_ref[...], axis=-1, keepdims=True) + rg_b2_ref[...]
    rgate = jax.nn.sigmoid(rlogit)
    out = x + rgate * o_b_ref[...]
    mu = jnp.mean(out, axis=-1, keepdims=True)
    d = out - mu
    var = jnp.mean(d * d, axis=-1, keepdims=True)
    out_ref[...] = d * jax.lax.rsqrt(var + 1e-5) * ln_g_ref[...] + ln_b_ref[...]

    a2 = _silu(jnp.dot(x, wg_w1_ref[...], preferred_element_type=jnp.float32)
               + wg_b1_ref[...])
    wlogit = jnp.sum(a2 * wg_w2t_ref[...], axis=-1, keepdims=True) + wg_b2_ref[...]
    wgate = jax.nn.sigmoid(wlogit)
    row = jax.lax.broadcasted_iota(jnp.int32, (_CBLK, _BLK), 0)
    col = jax.lax.broadcasted_iota(jnp.int32, (_CBLK, _BLK), 1)
    pool = jnp.where(col // _CR == row, 1.0 / _CR, 0.0)
    gmean = jnp.dot(pool, wgate, preferred_element_type=jnp.float32)

    xc = hsc_ref[...]
    h1 = _silu(jnp.dot(xc, comp_w1_ref[...], preferred_element_type=jnp.float32)
               + comp_b1_ref[...])
    comp = (jnp.dot(h1, comp_w2_ref[...], preferred_element_type=jnp.float32)
            + comp_b2_ref[...])
    bank_ref[...] = comp * gmean


def kernel(hidden_states, comp_w1, comp_b1, comp_w2, comp_b2,
           q_w, q_b, k_w, k_b, v_w, v_b, o_w, o_b,
           wg_w1, wg_b1, wg_w2, wg_b2, rg_w1, rg_b1, rg_w2, rg_b2,
           ln_g, ln_b):
    b, s, h = hidden_states.shape
    n_tok = b * s
    hs2 = hidden_states.reshape(n_tok, h)
    hsc = hidden_states.reshape(n_tok // _CR, _CR * h)

    rg_w1a = rg_w1[:h]
    rg_c1 = (o_b @ rg_w1[h:] + rg_b1).reshape(1, -1)

    grid = (n_tok // _BLK,)
    full = lambda arr: pl.BlockSpec(arr.shape, lambda i: (0,) * arr.ndim)

    w_args = (rg_w1a, rg_c1, rg_w2.reshape(1, -1), rg_b2.reshape(1, 1),
              wg_w1, wg_b1.reshape(1, -1),
              wg_w2.reshape(1, -1), wg_b2.reshape(1, 1),
              comp_w1, comp_b1.reshape(1, -1), comp_w2, comp_b2.reshape(1, -1),
              o_b.reshape(1, -1), ln_g.reshape(1, -1), ln_b.reshape(1, -1))

    out2, bank2 = pl.pallas_call(
        _fused_body,
        grid=grid,
        in_specs=[pl.BlockSpec((_BLK, h), lambda i: (i, 0)),
                  pl.BlockSpec((_CBLK, _CR * h), lambda i: (i, 0)),
                  *(full(a) for a in w_args)],
        out_specs=[pl.BlockSpec((_BLK, h), lambda i: (i, 0)),
                   pl.BlockSpec((_CBLK, h), lambda i: (i, 0))],
        out_shape=[jax.ShapeDtypeStruct((n_tok, h), jnp.float32),
                   jax.ShapeDtypeStruct((n_tok // _CR, h), jnp.float32)],
    )(hs2, hsc, *w_args)

    output = out2.reshape(b, s, h)
    memory_bank = bank2.reshape(b, s // _CR, h)
    memory_mask = jnp.ones((b, _MS), dtype=bool)
    return (output, memory_bank, memory_mask)


# speedup vs baseline: 1.4248x; 1.0522x over previous
import jax
import jax.numpy as jnp
from jax.experimental import pallas as pl

_B, _S, _H = 2, 2048, 1024
_MS, _CR = 512, 4
_BLK = 1024
_CBLK = _BLK // _CR


def _silu(x):
    return x * jax.nn.sigmoid(x)


def _fused_body(hs_ref, hsc_ref,
                rg_w1a_ref, rg_c1_ref, rg_w2t_ref, rg_b2_ref,
                wg_w1_ref, wg_b1_ref, wg_w2t_ref, wg_b2_ref,
                comp_w1_ref, comp_b1_ref, comp_w2_ref, comp_b2_ref,
                o_b_ref, ln_g_ref, ln_b_ref,
                out_ref, bank_ref):
    x = hs_ref[...]

    a1 = _silu(jnp.dot(x, rg_w1a_ref[...], preferred_element_type=jnp.float32)
               + rg_c1_ref[...])
    rlogit = jnp.sum(a1 * rg_w2t_ref[...], axis=-1, keepdims=True) + rg_b2_ref[...]
    rgate = jax.nn.sigmoid(rlogit)
    out = x + rgate * o_b_ref[...]
    mu = jnp.mean(out, axis=-1, keepdims=True)
    d = out - mu
    var = jnp.mean(d * d, axis=-1, keepdims=True)
    out_ref[...] = d * jax.lax.rsqrt(var + 1e-5) * ln_g_ref[...] + ln_b_ref[...]

    a2 = _silu(jnp.dot(x, wg_w1_ref[...], preferred_element_type=jnp.float32)
               + wg_b1_ref[...])
    wlogit = jnp.sum(a2 * wg_w2t_ref[...], axis=-1, keepdims=True) + wg_b2_ref[...]
    wgate = jax.nn.sigmoid(wlogit)
    row = jax.lax.broadcasted_iota(jnp.int32, (_CBLK, _BLK), 0)
    col = jax.lax.broadcasted_iota(jnp.int32, (_CBLK, _BLK), 1)
    pool = jnp.where(col // _CR == row, 1.0 / _CR, 0.0)
    gmean = jnp.dot(pool, wgate, preferred_element_type=jnp.float32)

    xc = hsc_ref[...]
    h1 = _silu(jnp.dot(xc, comp_w1_ref[...], preferred_element_type=jnp.float32)
               + comp_b1_ref[...])
    comp = (jnp.dot(h1, comp_w2_ref[...], preferred_element_type=jnp.float32)
            + comp_b2_ref[...])
    bank_ref[...] = comp * gmean


def kernel(hidden_states, comp_w1, comp_b1, comp_w2, comp_b2,
           q_w, q_b, k_w, k_b, v_w, v_b, o_w, o_b,
           wg_w1, wg_b1, wg_w2, wg_b2, rg_w1, rg_b1, rg_w2, rg_b2,
           ln_g, ln_b):
    b, s, h = hidden_states.shape
    n_tok = b * s
    hs2 = hidden_states.reshape(n_tok, h)
    hsc = hidden_states.reshape(n_tok // _CR, _CR * h)

    rg_w1a = rg_w1[:h]
    rg_c1 = (o_b @ rg_w1[h:] + rg_b1).reshape(1, -1)

    grid = (n_tok // _BLK,)
    full = lambda arr: pl.BlockSpec(arr.shape, lambda i: (0,) * arr.ndim)

    w_args = (rg_w1a, rg_c1, rg_w2.reshape(1, -1), rg_b2.reshape(1, 1),
              wg_w1, wg_b1.reshape(1, -1),
              wg_w2.reshape(1, -1), wg_b2.reshape(1, 1),
              comp_w1, comp_b1.reshape(1, -1), comp_w2, comp_b2.reshape(1, -1),
              o_b.reshape(1, -1), ln_g.reshape(1, -1), ln_b.reshape(1, -1))

    out2, bank2 = pl.pallas_call(
        _fused_body,
        grid=grid,
        in_specs=[pl.BlockSpec((_BLK, h), lambda i: (i, 0)),
                  pl.BlockSpec((_CBLK, _CR * h), lambda i: (i, 0)),
                  *(full(a) for a in w_args)],
        out_specs=[pl.BlockSpec((_BLK, h), lambda i: (i, 0)),
                   pl.BlockSpec((_CBLK, h), lambda i: (i, 0))],
        out_shape=[jax.ShapeDtypeStruct((n_tok, h), jnp.float32),
                   jax.ShapeDtypeStruct((n_tok // _CR, h), jnp.float32)],
    )(hs2, hsc, *w_args)

    output = out2.reshape(b, s, h)
    memory_bank = bank2.reshape(b, s // _CR, h)
    memory_mask = jnp.ones((b, _MS), dtype=bool)
    return (output, memory_bank, memory_mask)


# speedup vs baseline: 1.8098x; 1.2702x over previous
import jax
import jax.numpy as jnp
from jax.experimental import pallas as pl

_B, _S, _H = 2, 2048, 1024
_MS, _CR = 512, 4
_BLK = 1024
_CBLK = _BLK // _CR


def _silu(x):
    return x * jax.nn.sigmoid(x)


def _dot(a, b):
    return jnp.dot(a, b, preferred_element_type=jnp.float32)


def _fused_body(hs_ref,
                rg_w1_ref, rg_b1_ref, rg_w2_ref, rg_b2_ref,
                wg_w1_ref, wg_b1_ref, wg_w2_ref, wg_b2_ref,
                comp_w1_ref, comp_b1_ref, comp_w2_ref, comp_b2_ref,
                o_b_ref, ln_g_ref, ln_b_ref,
                out_ref, bank_ref):
    x = hs_ref[...]
    ob = o_b_ref[...][None, :]

    c1 = _dot(ob, rg_w1_ref[_H:, :]) + rg_b1_ref[...][None, :]
    a1 = _silu(_dot(x, rg_w1_ref[:_H, :]) + c1)
    rgate = jax.nn.sigmoid(_dot(a1, rg_w2_ref[...]) + rg_b2_ref[...][None, :])
    out = x + rgate * ob
    mu = jnp.mean(out, axis=-1, keepdims=True)
    d = out - mu
    var = jnp.mean(d * d, axis=-1, keepdims=True)
    out_ref[...] = (d * jax.lax.rsqrt(var + 1e-5) * ln_g_ref[...][None, :]
                    + ln_b_ref[...][None, :])

    a2 = _silu(_dot(x, wg_w1_ref[...]) + wg_b1_ref[...][None, :])
    wgate = jax.nn.sigmoid(_dot(a2, wg_w2_ref[...]) + wg_b2_ref[...][None, :])
    row = jax.lax.broadcasted_iota(jnp.int32, (_CBLK, _BLK), 0)
    col = jax.lax.broadcasted_iota(jnp.int32, (_CBLK, _BLK), 1)
    pool = jnp.where(col // _CR == row, 1.0 / _CR, 0.0)
    gmean = _dot(pool, wgate)

    x4 = x.reshape(_CBLK, _CR, _H)
    acc = comp_b1_ref[...][None, :].astype(jnp.float32) + jnp.zeros((_CBLK, _H), jnp.float32)
    for j in range(_CR):
        acc = acc + _dot(x4[:, j, :], comp_w1_ref[pl.ds(j * _H, _H), :])
    h1 = _silu(acc)
    comp = _dot(h1, comp_w2_ref[...]) + comp_b2_ref[...][None, :]
    bank_ref[...] = comp * gmean


def kernel(hidden_states, comp_w1, comp_b1, comp_w2, comp_b2,
           q_w, q_b, k_w, k_b, v_w, v_b, o_w, o_b,
           wg_w1, wg_b1, wg_w2, wg_b2, rg_w1, rg_b1, rg_w2, rg_b2,
           ln_g, ln_b):
    b, s, h = hidden_states.shape
    n_tok = b * s
    hs2 = hidden_states.reshape(n_tok, h)

    grid = (n_tok // _BLK,)
    full = lambda arr: pl.BlockSpec(arr.shape, lambda i: (0,) * arr.ndim)

    w_args = (rg_w1, rg_b1, rg_w2, rg_b2,
              wg_w1, wg_b1, wg_w2, wg_b2,
              comp_w1, comp_b1, comp_w2, comp_b2,
              o_b, ln_g, ln_b)

    out2, bank2 = pl.pallas_call(
        _fused_body,
        grid=grid,
        in_specs=[pl.BlockSpec((_BLK, h), lambda i: (i, 0)),
                  *(full(a) for a in w_args)],
        out_specs=[pl.BlockSpec((_BLK, h), lambda i: (i, 0)),
                   pl.BlockSpec((_CBLK, h), lambda i: (i, 0))],
        out_shape=[jax.ShapeDtypeStruct((n_tok, h), jnp.float32),
                   jax.ShapeDtypeStruct((n_tok // _CR, h), jnp.float32)],
    )(hs2, *w_args)

    output = out2.reshape(b, s, h)
    memory_bank = bank2.reshape(b, s // _CR, h)
    memory_mask = jnp.ones((b, _MS), dtype=bool)
    return (output, memory_bank, memory_mask)
